# baseline (speedup 1.0000x reference)
import functools

import jax
import jax.numpy as jnp
from jax import lax
from jax.experimental import pallas as pl
from jax.experimental.pallas import tpu as pltpu
from jax.experimental.pallas import tpu_sc as plsc

HIDDEN = 128
NC, NS = 2, 16
NW = NC * NS
CHUNK = 128
NBUF = 4
XBUF = 2
DBUF = 2


def _make_lookup(B):
    b_per_w = B // NW
    n_chunks = b_per_w // CHUNK
    n_groups = n_chunks // 2
    mesh = plsc.VectorSubcoreMesh(
        core_axis_name="c", subcore_axis_name="s", num_cores=NC, num_subcores=NS
    )

    @functools.partial(
        pl.kernel,
        out_type=jax.ShapeDtypeStruct((B, HIDDEN), jnp.float32),
        mesh=mesh,
        scratch_types=[
            pltpu.VMEM((n_chunks, CHUNK), jnp.int32),
            pltpu.VMEM((NBUF, CHUNK, HIDDEN), jnp.float32),
            pltpu.VMEM_SHARED((NS, XBUF, CHUNK, HIDDEN), jnp.float32),
            pltpu.SemaphoreType.DMA((NBUF,)),
            pltpu.SemaphoreType.DMA((XBUF,)),
            pltpu.SemaphoreType.DMA((XBUF,)),
            pltpu.SemaphoreType.DMA((DBUF,)),
        ],
    )
    def lookup(idx_hbm, table_hbm, out_hbm, idx_v, rows_v, shr, gsem, xsem, ssem, dsem):
        wid = lax.axis_index("s") * NC + lax.axis_index("c")
        sid = lax.axis_index("s")
        pltpu.sync_copy(idx_hbm.at[pl.ds(wid * n_chunks, n_chunks)], idx_v)
        base = wid * b_per_w

        def gather(c):
            pltpu.async_copy(
                table_hbm.at[idx_v.at[c]],
                rows_v.at[lax.rem(c, NBUF)],
                gsem.at[lax.rem(c, NBUF)],
            )

        def wait_gather(c):
            s = lax.rem(c, NBUF)
            pltpu.make_async_copy(
                table_hbm.at[idx_v.at[s]], rows_v.at[s], gsem.at[s]
            ).wait()

        def spmem_store_of(g):
            x = lax.rem(g, XBUF)
            pltpu.make_async_copy(rows_v.at[0], shr.at[sid, x], xsem.at[x]).wait()
            pltpu.async_copy(
                shr.at[sid, x],
                out_hbm.at[pl.ds(base + 2 * g * CHUNK, CHUNK)],
                ssem.at[x],
            )

        gather(0)
        gather(1)

        def body(g, carry):
            xslot = lax.rem(g, XBUF)
            dslot = lax.rem(g, DBUF)

            @pl.when(g >= 1)
            def _():
                spmem_store_of(g - 1)

            @pl.when(g + 1 < n_groups)
            def _():
                @pl.when(g >= 1)
                def _():
                    pltpu.make_async_copy(
                        rows_v.at[0],
                        out_hbm.at[pl.ds(0, CHUNK)],
                        dsem.at[lax.rem(g - 1, DBUF)],
                    ).wait()

                gather(2 * g + 2)
                gather(2 * g + 3)

            wait_gather(2 * g)

            @pl.when(g >= XBUF)
            def _():
                pltpu.make_async_copy(
                    shr.at[sid, xslot], out_hbm.at[pl.ds(0, CHUNK)], ssem.at[xslot]
                ).wait()

            pltpu.async_copy(
                rows_v.at[lax.rem(2 * g, NBUF)], shr.at[sid, xslot], xsem.at[xslot]
            )

            wait_gather(2 * g + 1)
            pltpu.async_copy(
                rows_v.at[lax.rem(2 * g + 1, NBUF)],
                out_hbm.at[pl.ds(base + (2 * g + 1) * CHUNK, CHUNK)],
                dsem.at[dslot],
            )
            return carry

        lax.fori_loop(0, n_groups, body, 0)

        spmem_store_of(n_groups - 1)
        for b in range(XBUF):
            pltpu.make_async_copy(
                shr.at[sid, b], out_hbm.at[pl.ds(0, CHUNK)], ssem.at[b]
            ).wait()
        for b in range(DBUF):
            pltpu.make_async_copy(
                rows_v.at[0], out_hbm.at[pl.ds(0, CHUNK)], dsem.at[b]
            ).wait()

    return lookup


def kernel(token_ids, table):
    B_, S_ = token_ids.shape
    flat = jnp.reshape(token_ids, (-1, CHUNK)).astype(jnp.int32)
    out = _make_lookup(B_ * S_)(flat, table)
    return jnp.reshape(out, (B_, S_, HIDDEN))


# speedup vs baseline: 1.0098x; 1.0098x over previous
import functools

import jax
import jax.numpy as jnp
from jax import lax
from jax.experimental import pallas as pl
from jax.experimental.pallas import tpu as pltpu
from jax.experimental.pallas import tpu_sc as plsc

HIDDEN = 128
NC, NS = 2, 16
NW = NC * NS
CHUNK = 128
NBUF = 4
LOOK = 3
XBUF = 2


def _make_lookup(B):
    b_per_w = B // NW
    n_chunks = b_per_w // CHUNK
    mesh = plsc.VectorSubcoreMesh(
        core_axis_name="c", subcore_axis_name="s", num_cores=NC, num_subcores=NS
    )

    @functools.partial(
        pl.kernel,
        out_type=jax.ShapeDtypeStruct((B, HIDDEN), jnp.float32),
        mesh=mesh,
        scratch_types=[
            pltpu.VMEM((n_chunks, CHUNK), jnp.int32),
            pltpu.VMEM((NBUF, CHUNK, HIDDEN), jnp.float32),
            pltpu.VMEM_SHARED((NS, XBUF, CHUNK, HIDDEN), jnp.float32),
            pltpu.SemaphoreType.DMA((NBUF,)),
            pltpu.SemaphoreType.DMA((XBUF,)),
            pltpu.SemaphoreType.DMA((XBUF,)),
        ],
    )
    def lookup(idx_hbm, table_hbm, out_hbm, idx_v, rows_v, shr, gsem, xsem, ssem):
        wid = lax.axis_index("s") * NC + lax.axis_index("c")
        sid = lax.axis_index("s")
        pltpu.sync_copy(idx_hbm.at[pl.ds(wid * n_chunks, n_chunks)], idx_v)
        base = wid * b_per_w

        for j in range(LOOK):
            pltpu.async_copy(table_hbm.at[idx_v.at[j]], rows_v.at[j], gsem.at[j])

        def store_of(c):
            pslot = lax.rem(c, XBUF)
            pltpu.make_async_copy(
                rows_v.at[0], shr.at[sid, pslot], xsem.at[pslot]
            ).wait()
            pltpu.async_copy(
                shr.at[sid, pslot],
                out_hbm.at[pl.ds(base + c * CHUNK, CHUNK)],
                ssem.at[pslot],
            )

        def body(i, carry):
            slot = lax.rem(i, NBUF)
            xslot = lax.rem(i, XBUF)

            @pl.when(i >= 1)
            def _():
                store_of(i - 1)

            j = i + LOOK

            @pl.when(j < n_chunks)
            def _():
                pltpu.async_copy(
                    table_hbm.at[idx_v.at[j]],
                    rows_v.at[lax.rem(j, NBUF)],
                    gsem.at[lax.rem(j, NBUF)],
                )

            pltpu.make_async_copy(
                table_hbm.at[idx_v.at[slot]], rows_v.at[slot], gsem.at[slot]
            ).wait()

            @pl.when(i >= XBUF)
            def _():
                pltpu.make_async_copy(
                    shr.at[sid, xslot], out_hbm.at[pl.ds(0, CHUNK)], ssem.at[xslot]
                ).wait()

            pltpu.async_copy(rows_v.at[slot], shr.at[sid, xslot], xsem.at[xslot])
            return carry

        lax.fori_loop(0, n_chunks, body, 0)

        store_of(n_chunks - 1)
        for b in range(XBUF):
            pltpu.make_async_copy(
                shr.at[sid, b], out_hbm.at[pl.ds(0, CHUNK)], ssem.at[b]
            ).wait()

    return lookup


def kernel(token_ids, table):
    B_, S_ = token_ids.shape
    flat = jnp.reshape(token_ids, (-1, CHUNK)).astype(jnp.int32)
    out = _make_lookup(B_ * S_)(flat, table)
    return jnp.reshape(out, (B_, S_, HIDDEN))
